# baseline (speedup 1.0000x reference)
import functools

import jax
import jax.numpy as jnp
from jax import lax
from jax.experimental import pallas as pl
from jax.experimental.pallas import tpu as pltpu
from jax.experimental.pallas import tpu_sc as plsc

D = 128
BATCH = 4096
HIST = 200
B_TOTAL = BATCH * HIST
NC, NS = 2, 16
NW = NC * NS
PER_W = B_TOTAL // NW
CHUNK = 128
N_CHUNKS = PER_W // CHUNK
NBUF = 5
LAG = 2


def _gather_sc(seq_flat, table):
    mesh = plsc.VectorSubcoreMesh(core_axis_name="c", subcore_axis_name="s")

    @functools.partial(
        pl.kernel,
        mesh=mesh,
        out_type=jax.ShapeDtypeStruct((B_TOTAL, D), jnp.float32),
        scratch_types=[
            pltpu.VMEM((PER_W,), jnp.int32),
            pltpu.VMEM((NBUF, CHUNK, D), jnp.float32),
        ]
        + [pltpu.SemaphoreType.DMA] * (2 * NBUF),
    )
    def k(seq_hbm, table_hbm, out_hbm, idx_v, rows_v, *sems):
        gsems, osems = sems[:NBUF], sems[NBUF:]
        wid = lax.axis_index("s") * NC + lax.axis_index("c")
        base = wid * PER_W
        pltpu.sync_copy(seq_hbm.at[pl.ds(base, PER_W)], idx_v)

        def gather(chunk, slot):
            pltpu.async_copy(
                table_hbm.at[idx_v.at[pl.ds(chunk * CHUNK, CHUNK)]],
                rows_v.at[slot],
                gsems[slot],
            )

        def wait_gather(chunk, slot):
            pltpu.make_async_copy(
                table_hbm.at[idx_v.at[pl.ds(chunk * CHUNK, CHUNK)]],
                rows_v.at[slot],
                gsems[slot],
            ).wait()

        def writeout(chunk, slot):
            pltpu.async_copy(
                rows_v.at[slot],
                out_hbm.at[pl.ds(base + chunk * CHUNK, CHUNK)],
                osems[slot],
            )

        def wait_writeout(chunk, slot):
            pltpu.make_async_copy(
                rows_v.at[slot],
                out_hbm.at[pl.ds(base + chunk * CHUNK, CHUNK)],
                osems[slot],
            ).wait()

        for b in range(NBUF):
            gather(b, b)

        def body(g, _):
            for b in range(NBUF):
                j = g * NBUF + b
                wait_gather(j, b)
                writeout(j, b)
                jr = j - LAG
                c = (b - LAG) % NBUF

                @pl.when((jr >= 0) & (jr + NBUF < N_CHUNKS))
                def _():
                    wait_writeout(jr, c)
                    gather(jr + NBUF, c)

            return 0

        lax.fori_loop(0, N_CHUNKS // NBUF, body, 0)

        for j in range(N_CHUNKS - NBUF, N_CHUNKS):
            wait_writeout(j, j % NBUF)

    return k(seq_flat, table)


@jax.jit
def kernel(seq, table):
    out = _gather_sc(seq.reshape(-1), table)
    return out.reshape(BATCH, HIST, D)
